# baseline (speedup 1.0000x reference)
import jax
import jax.numpy as jnp
from jax.experimental import pallas as pl
from jax.experimental.pallas import tpu as pltpu


def _tc_body(x_ref, o_ref, buf, si, so):
    ins = []
    for k in range(18):
        d0, j = divmod(k, 9)
        ins.append(
            pltpu.make_async_copy(x_ref.at[d0, 2 * j], buf.at[d0, j], si.at[k])
        )
    for j in range(9):
        for d0 in range(2):
            ins[d0 * 9 + j].start()
    outs = []
    for g in range(6):
        d0, jg = divmod(g, 3)
        for k in range(3 * g, 3 * g + 3):
            ins[k].wait()
        oc = pltpu.make_async_copy(
            buf.at[d0, pl.ds(3 * jg, 3)], o_ref.at[d0, pl.ds(3 * jg, 3)], so.at[g]
        )
        oc.start()
        outs.append(oc)
    for oc in outs:
        oc.wait()


@jax.jit
def _dilated_panels_tc(y):
    return pl.pallas_call(
        _tc_body,
        in_specs=[pl.BlockSpec(memory_space=pltpu.MemorySpace.HBM)],
        out_specs=pl.BlockSpec(memory_space=pltpu.MemorySpace.HBM),
        out_shape=jax.ShapeDtypeStruct((2, 9, 8, 10000), jnp.int32),
        scratch_shapes=[
            pltpu.VMEM((2, 9, 8, 10000), jnp.int32),
            pltpu.SemaphoreType.DMA((18,)),
            pltpu.SemaphoreType.DMA((6,)),
        ],
        compiler_params=pltpu.CompilerParams(
            vmem_limit_bytes=52 * 1024 * 1024,
        ),
    )(y)


def kernel(edge_index):
    y = jnp.transpose(edge_index, (0, 3, 1, 2))
    out_t = _dilated_panels_tc(y)
    return jnp.transpose(out_t, (0, 2, 3, 1))


# speedup vs baseline: 1.0530x; 1.0530x over previous
import jax
import jax.numpy as jnp
from jax.experimental import pallas as pl
from jax.experimental.pallas import tpu as pltpu


def _tc_body(x_ref, o_ref, buf, si, so):
    ins = []
    for k in range(18):
        d0, j = divmod(k, 9)
        ins.append(
            pltpu.make_async_copy(x_ref.at[d0, 2 * j], buf.at[d0, j], si.at[k])
        )
    for c in ins:
        c.start()
    outs = []
    for g in range(6):
        d0, jg = divmod(g, 3)
        for k in range(3 * g, 3 * g + 3):
            ins[k].wait()
        oc = pltpu.make_async_copy(
            buf.at[d0, pl.ds(3 * jg, 3)], o_ref.at[d0, pl.ds(3 * jg, 3)], so.at[g]
        )
        oc.start()
        outs.append(oc)
    for oc in outs:
        oc.wait()


@jax.jit
def _dilated_panels_tc(y):
    return pl.pallas_call(
        _tc_body,
        in_specs=[pl.BlockSpec(memory_space=pltpu.MemorySpace.HBM)],
        out_specs=pl.BlockSpec(memory_space=pltpu.MemorySpace.HBM),
        out_shape=jax.ShapeDtypeStruct((2, 9, 8, 10000), jnp.int32),
        scratch_shapes=[
            pltpu.VMEM((2, 9, 8, 10000), jnp.int32),
            pltpu.SemaphoreType.DMA((18,)),
            pltpu.SemaphoreType.DMA((6,)),
        ],
        compiler_params=pltpu.CompilerParams(
            vmem_limit_bytes=52 * 1024 * 1024,
        ),
    )(y)


def kernel(edge_index):
    y = jnp.transpose(edge_index, (0, 3, 1, 2))
    out_t = _dilated_panels_tc(y)
    return jnp.transpose(out_t, (0, 2, 3, 1))
